# Initial kernel scaffold; baseline (speedup 1.0000x reference)
import jax
import jax.numpy as jnp
from jax.experimental import pallas as pl


def kernel(lt_ids_0, lt_ids_1, lt_ids_2, lt_ids_3, lt_ids_4, lt_ids_5, gt_ids_0, gt_ids_1, gt_ids_2, gt_ids_3, gt_ids_4, show_ids, lt_tables, gt_tables, show_table, lin_W, lin_b):
    raise NotImplementedError("write your pallas kernel here")


# speedup vs baseline: 11.8834x; 11.8834x over previous
import functools

import jax
import jax.numpy as jnp
from jax import lax
from jax.experimental import pallas as pl
from jax.experimental.pallas import tpu as pltpu
from jax.experimental.pallas import tpu_sc as plsc

B = 4096
L = 50
D = 64
NUM_BAG = 11
NUM_TAB = NUM_BAG + 1
NC, NS = 2, 16
NW = NC * NS
BW = B // NW
CH = 8
NCH = BW // CH
ROWS = CH * L


def _sc_embed(tables, ids_flat, show_table, show_ids):
    mesh = plsc.VectorSubcoreMesh(
        core_axis_name="c", subcore_axis_name="s", num_cores=NC, num_subcores=NS
    )

    @functools.partial(
        pl.kernel,
        out_type=jax.ShapeDtypeStruct((B, NUM_TAB * D), jnp.float32),
        mesh=mesh,
        scratch_types=[
            pltpu.VMEM((ROWS,), jnp.int32),
            pltpu.VMEM((ROWS, D), jnp.float32),
            pltpu.VMEM((BW, D), jnp.float32),
            pltpu.VMEM((BW, 2 * D), jnp.float32),
            pltpu.SemaphoreType.DMA,
        ],
        compiler_params=pltpu.CompilerParams(use_tc_tiling_on_sc=False),
    )
    def k(*refs):
        t_refs = refs[:NUM_BAG]
        i_refs = refs[NUM_BAG:2 * NUM_BAG]
        show_t, show_i, out, idx_v, rows_v, show_v, acc_v, sem = refs[2 * NUM_BAG:]

        wid = lax.axis_index("s") * NC + lax.axis_index("c")
        base = wid * BW

        for pair in range(NUM_TAB // 2):
            for sub in range(2):
                t = 2 * pair + sub
                if t == NUM_BAG:
                    pltpu.sync_copy(show_i.at[pl.ds(base, BW)],
                                    idx_v.at[pl.ds(0, BW)])
                    pltpu.async_copy(show_t.at[idx_v.at[pl.ds(0, BW)]],
                                     show_v, sem).wait()

                    def show_row(j, _):
                        for q in range(4):
                            acc_v[j, pl.ds(sub * D + 16 * q, 16)] = (
                                show_v[j, pl.ds(16 * q, 16)])
                        return 0

                    lax.fori_loop(0, BW, show_row, 0)
                    continue

                def chunk_body(c, _, t=t, sub=sub):
                    pltpu.sync_copy(
                        i_refs[t].at[pl.ds(base * L + c * ROWS, ROWS)], idx_v)
                    pltpu.async_copy(t_refs[t].at[idx_v], rows_v, sem).wait()

                    def bag_body(j, _, sub=sub):
                        row0 = j * L
                        accs = tuple(rows_v[row0, pl.ds(16 * q, 16)]
                                     for q in range(4))

                        def l_body(l, a):
                            return tuple(
                                jnp.maximum(a[q],
                                            rows_v[row0 + l, pl.ds(16 * q, 16)])
                                for q in range(4)
                            )

                        accs = lax.fori_loop(1, L, l_body, accs)
                        for q in range(4):
                            acc_v[c * CH + j, pl.ds(sub * D + 16 * q, 16)] = accs[q]
                        return 0

                    lax.fori_loop(0, CH, bag_body, 0)
                    return 0

                lax.fori_loop(0, NCH, chunk_body, 0)
            pltpu.sync_copy(acc_v,
                            out.at[pl.ds(base, BW), pl.ds(pair * 2 * D, 2 * D)])

    return k(*tables, *ids_flat, show_table, show_ids)


def _tc_heads(emb, lin_W, lin_b):
    def mm(emb_ref, w_ref, b_ref, out_ref):
        x = emb_ref[...]
        for i in range(lin_W.shape[0]):
            out_ref[i] = (
                jnp.dot(x, w_ref[i], preferred_element_type=jnp.float32)
                + b_ref[i][None, :]
            )

    return pl.pallas_call(
        mm,
        out_shape=jax.ShapeDtypeStruct((lin_W.shape[0], B, 5), jnp.float32),
    )(emb, lin_W, lin_b)


def kernel(lt_ids_0, lt_ids_1, lt_ids_2, lt_ids_3, lt_ids_4, lt_ids_5,
           gt_ids_0, gt_ids_1, gt_ids_2, gt_ids_3, gt_ids_4,
           show_ids, lt_tables, gt_tables, show_table, lin_W, lin_b):
    lt_ids = [lt_ids_0, lt_ids_1, lt_ids_2, lt_ids_3, lt_ids_4, lt_ids_5]
    gt_ids = [gt_ids_0, gt_ids_1, gt_ids_2, gt_ids_3, gt_ids_4]
    tables = [lt_tables[i] for i in range(6)] + [gt_tables[i] for i in range(5)]
    ids_flat = [x.reshape(-1) for x in lt_ids + gt_ids]
    emb = _sc_embed(tables, ids_flat, show_table, show_ids)
    return _tc_heads(emb, lin_W, lin_b)


# speedup vs baseline: 18.3994x; 1.5483x over previous
import functools

import jax
import jax.numpy as jnp
from jax import lax
from jax.experimental import pallas as pl
from jax.experimental.pallas import tpu as pltpu
from jax.experimental.pallas import tpu_sc as plsc

B = 4096
L = 50
D = 64
NUM_BAG = 11
NUM_TAB = NUM_BAG + 1
NC, NS = 2, 16
NW = NC * NS
BW = B // NW
CH = 8
NCH = BW // CH
ROWS = CH * L


def _sc_embed(tables, ids_flat, show_table, show_ids):
    mesh = plsc.VectorSubcoreMesh(
        core_axis_name="c", subcore_axis_name="s", num_cores=NC, num_subcores=NS
    )

    @functools.partial(
        pl.kernel,
        out_type=jax.ShapeDtypeStruct((B, NUM_TAB * D), jnp.float32),
        mesh=mesh,
        scratch_types=[
            pltpu.VMEM((BW * L,), jnp.int32),
            pltpu.VMEM((ROWS, D), jnp.float32),
            pltpu.VMEM((ROWS, D), jnp.float32),
            pltpu.VMEM((BW, D), jnp.float32),
            pltpu.SemaphoreType.DMA,
            pltpu.SemaphoreType.DMA,
        ],
        compiler_params=pltpu.CompilerParams(use_tc_tiling_on_sc=False),
    )
    def k(*refs):
        t_refs = refs[:NUM_BAG]
        i_refs = refs[NUM_BAG:2 * NUM_BAG]
        show_t, show_i, out, idx_all, buf_a, buf_b, acc_v, sem_a, sem_b = (
            refs[2 * NUM_BAG:])

        wid = lax.axis_index("s") * NC + lax.axis_index("c")
        base = wid * BW

        def gather_start(tab, c, buf, sem):
            pltpu.async_copy(tab.at[idx_all.at[pl.ds(c * ROWS, ROWS)]],
                             buf, sem)

        def gather_wait(tab, c, buf, sem):
            pltpu.make_async_copy(tab.at[idx_all.at[pl.ds(c * ROWS, ROWS)]],
                                  buf, sem).wait()

        def compute_chunk(c, buf):
            def bag_body(j, _):
                row0 = j * L
                accs = tuple(buf[row0, pl.ds(16 * q, 16)] for q in range(4))

                def l_body(i, a):
                    r = row0 + 1 + 2 * i
                    a = tuple(jnp.maximum(a[q], buf[r, pl.ds(16 * q, 16)])
                              for q in range(4))
                    return tuple(jnp.maximum(a[q], buf[r + 1, pl.ds(16 * q, 16)])
                                 for q in range(4))

                accs = lax.fori_loop(0, (L - 2) // 2, l_body, accs)
                accs = tuple(jnp.maximum(accs[q],
                                         buf[row0 + L - 1, pl.ds(16 * q, 16)])
                             for q in range(4))
                for q in range(4):
                    acc_v[c * CH + j, pl.ds(16 * q, 16)] = accs[q]
                return 0

            lax.fori_loop(0, CH, bag_body, 0)

        for t in range(NUM_BAG):
            pltpu.sync_copy(i_refs[t].at[pl.ds(base * L, BW * L)], idx_all)
            gather_start(t_refs[t], 0, buf_a, sem_a)
            gather_start(t_refs[t], 1, buf_b, sem_b)

            def pipe(i, _, t=t):
                for p, (buf, sem) in enumerate(((buf_a, sem_a), (buf_b, sem_b))):
                    c = 2 * i + p
                    gather_wait(t_refs[t], c, buf, sem)
                    compute_chunk(c, buf)

                    @pl.when(c + 2 < NCH)
                    def _(c=c, buf=buf, sem=sem, t=t):
                        gather_start(t_refs[t], c + 2, buf, sem)
                return 0

            lax.fori_loop(0, NCH // 2, pipe, 0)
            pltpu.sync_copy(acc_v, out.at[pl.ds(base, BW), pl.ds(t * D, D)])

        pltpu.sync_copy(show_i.at[pl.ds(base, BW)], idx_all.at[pl.ds(0, BW)])
        pltpu.async_copy(show_t.at[idx_all.at[pl.ds(0, BW)]], acc_v,
                         sem_a).wait()
        pltpu.sync_copy(acc_v, out.at[pl.ds(base, BW), pl.ds(NUM_BAG * D, D)])

    return k(*tables, *ids_flat, show_table, show_ids)


def _tc_heads(emb, lin_W, lin_b):
    def mm(emb_ref, w_ref, b_ref, out_ref):
        x = emb_ref[...]
        for i in range(lin_W.shape[0]):
            out_ref[i] = (
                jnp.dot(x, w_ref[i], preferred_element_type=jnp.float32)
                + b_ref[i][None, :]
            )

    return pl.pallas_call(
        mm,
        out_shape=jax.ShapeDtypeStruct((lin_W.shape[0], B, 5), jnp.float32),
    )(emb, lin_W, lin_b)


def kernel(lt_ids_0, lt_ids_1, lt_ids_2, lt_ids_3, lt_ids_4, lt_ids_5,
           gt_ids_0, gt_ids_1, gt_ids_2, gt_ids_3, gt_ids_4,
           show_ids, lt_tables, gt_tables, show_table, lin_W, lin_b):
    lt_ids = [lt_ids_0, lt_ids_1, lt_ids_2, lt_ids_3, lt_ids_4, lt_ids_5]
    gt_ids = [gt_ids_0, gt_ids_1, gt_ids_2, gt_ids_3, gt_ids_4]
    tables = [lt_tables[i] for i in range(6)] + [gt_tables[i] for i in range(5)]
    ids_flat = [x.reshape(-1) for x in lt_ids + gt_ids]
    emb = _sc_embed(tables, ids_flat, show_table, show_ids)
    return _tc_heads(emb, lin_W, lin_b)


# speedup vs baseline: 19.6774x; 1.0695x over previous
import functools

import jax
import jax.numpy as jnp
from jax import lax
from jax.experimental import pallas as pl
from jax.experimental.pallas import tpu as pltpu
from jax.experimental.pallas import tpu_sc as plsc

B = 4096
L = 50
D = 64
V = 21000
NUM_LT = 6
NUM_GT = 5
NUM_BAG = NUM_LT + NUM_GT
NUM_TAB = NUM_BAG + 1
NC, NS = 2, 16
NW = NC * NS
BW = B // NW
CH = 16
NCH = BW // CH
ROWS = CH * L


def _sc_embed(lt_tab, gt_tab, lt_ids, gt_ids, show_tab, show_ids):
    mesh = plsc.VectorSubcoreMesh(
        core_axis_name="c", subcore_axis_name="s", num_cores=NC, num_subcores=NS
    )

    @functools.partial(
        pl.kernel,
        out_type=jax.ShapeDtypeStruct((B, NUM_TAB * D), jnp.bfloat16),
        mesh=mesh,
        scratch_types=[
            pltpu.VMEM((BW * L,), jnp.int32),
            pltpu.VMEM((ROWS, D), jnp.bfloat16),
            pltpu.VMEM((ROWS, D), jnp.bfloat16),
            pltpu.VMEM((BW, D), jnp.bfloat16),
            pltpu.SemaphoreType.DMA,
            pltpu.SemaphoreType.DMA,
        ],
        compiler_params=pltpu.CompilerParams(use_tc_tiling_on_sc=False),
    )
    def k(lt_t, gt_t, lt_i, gt_i, show_t, show_i, out,
          idx_all, buf_a, buf_b, acc_v, sem_a, sem_b):
        wid = lax.axis_index("s") * NC + lax.axis_index("c")
        base = wid * BW

        def gather_start(tab, c, buf, sem):
            pltpu.async_copy(tab.at[idx_all.at[pl.ds(c * ROWS, ROWS)]],
                             buf, sem)

        def gather_wait(tab, c, buf, sem):
            pltpu.make_async_copy(tab.at[idx_all.at[pl.ds(c * ROWS, ROWS)]],
                                  buf, sem).wait()

        def compute_chunk(c, buf):
            def bag_body(j, _):
                row0 = j * L
                accs = tuple(buf[row0, pl.ds(32 * q, 32)] for q in range(2))

                def l_body(i, a):
                    r = row0 + 1 + 2 * i
                    a = tuple(jnp.maximum(a[q], buf[r, pl.ds(32 * q, 32)])
                              for q in range(2))
                    return tuple(jnp.maximum(a[q], buf[r + 1, pl.ds(32 * q, 32)])
                                 for q in range(2))

                accs = lax.fori_loop(0, (L - 2) // 2, l_body, accs)
                accs = tuple(jnp.maximum(accs[q],
                                         buf[row0 + L - 1, pl.ds(32 * q, 32)])
                             for q in range(2))
                for q in range(2):
                    acc_v[c * CH + j, pl.ds(32 * q, 32)] = accs[q]
                return 0

            lax.fori_loop(0, CH, bag_body, 0)

        for t in range(NUM_BAG):
            tab = lt_t if t < NUM_LT else gt_t
            ids = lt_i if t < NUM_LT else gt_i
            toff = t * B * L if t < NUM_LT else (t - NUM_LT) * B * L
            pltpu.sync_copy(ids.at[pl.ds(toff + base * L, BW * L)], idx_all)
            gather_start(tab, 0, buf_a, sem_a)
            gather_start(tab, 1, buf_b, sem_b)

            def pipe(i, _, tab=tab):
                for p, (buf, sem) in enumerate(((buf_a, sem_a), (buf_b, sem_b))):
                    c = 2 * i + p
                    gather_wait(tab, c, buf, sem)
                    compute_chunk(c, buf)

                    @pl.when(c + 2 < NCH)
                    def _(c=c, buf=buf, sem=sem, tab=tab):
                        gather_start(tab, c + 2, buf, sem)
                return 0

            lax.fori_loop(0, NCH // 2, pipe, 0)
            pltpu.sync_copy(acc_v, out.at[pl.ds(base, BW), pl.ds(t * D, D)])

        pltpu.sync_copy(show_i.at[pl.ds(base, BW)], idx_all.at[pl.ds(0, BW)])
        pltpu.async_copy(show_t.at[idx_all.at[pl.ds(0, BW)]], acc_v,
                         sem_a).wait()
        pltpu.sync_copy(acc_v, out.at[pl.ds(base, BW), pl.ds(NUM_BAG * D, D)])

    return k(lt_tab, gt_tab, lt_ids, gt_ids, show_tab, show_ids)


def _tc_heads(emb, lin_W, lin_b):
    def mm(emb_ref, w_ref, b_ref, out_ref):
        x = emb_ref[...].astype(jnp.float32)
        for i in range(lin_W.shape[0]):
            out_ref[i] = (
                jnp.dot(x, w_ref[i], preferred_element_type=jnp.float32)
                + b_ref[i][None, :]
            )

    return pl.pallas_call(
        mm,
        out_shape=jax.ShapeDtypeStruct((lin_W.shape[0], B, 5), jnp.float32),
    )(emb, lin_W, lin_b)


def kernel(lt_ids_0, lt_ids_1, lt_ids_2, lt_ids_3, lt_ids_4, lt_ids_5,
           gt_ids_0, gt_ids_1, gt_ids_2, gt_ids_3, gt_ids_4,
           show_ids, lt_tables, gt_tables, show_table, lin_W, lin_b):
    lt_ids = jnp.stack([lt_ids_0, lt_ids_1, lt_ids_2, lt_ids_3, lt_ids_4,
                        lt_ids_5]).reshape(NUM_LT, B * L)
    gt_ids = jnp.stack([gt_ids_0, gt_ids_1, gt_ids_2, gt_ids_3,
                        gt_ids_4]).reshape(NUM_GT, B * L)
    lt_off = (jnp.arange(NUM_LT, dtype=jnp.int32) * V)[:, None]
    gt_off = (jnp.arange(NUM_GT, dtype=jnp.int32) * V)[:, None]
    emb = _sc_embed(
        lt_tables.astype(jnp.bfloat16).reshape(NUM_LT * V, D),
        gt_tables.astype(jnp.bfloat16).reshape(NUM_GT * V, D),
        (lt_ids + lt_off).reshape(-1),
        (gt_ids + gt_off).reshape(-1),
        show_table.astype(jnp.bfloat16),
        show_ids,
    )
    return _tc_heads(emb, lin_W, lin_b)
